# Initial kernel scaffold; baseline (speedup 1.0000x reference)
import jax
import jax.numpy as jnp
from jax.experimental import pallas as pl


def kernel(x, edge_index, W1, b1, g1, beta1, a1, W2, b2, g2, beta2, a2, We2d, mask_token, Wd, bd):
    raise NotImplementedError("write your pallas kernel here")


# speedup vs baseline: 7.1459x; 7.1459x over previous
import functools

import jax
import jax.numpy as jnp
from jax import lax
from jax.experimental import pallas as pl
from jax.experimental.pallas import tpu as pltpu
import jax.experimental.pallas.tpu_sc as plsc

_N = 10000
_E = 320000
_D = 128
_NUM_MASK = 5000
_EPS_LN = 1e-5

_NP = 10240
_RPS = _NP // 16
_EC = _E // 128
_SP_SUB = 80
_SP_CORE = _SP_SUB * 16
_DG_SUB = 160


def _mask_vec():
    perm = jax.random.permutation(jax.random.key(1), _N)
    return jnp.zeros((_N, 1), jnp.float32).at[perm[:_NUM_MASK], 0].set(1.0)


_sc_mesh = plsc.VectorSubcoreMesh(core_axis_name="c", subcore_axis_name="s")
_sc_params = pltpu.CompilerParams(use_tc_tiling_on_sc=False)


def _load_idx_chunk(src2d, j, dst1):
    for k in range(8):
        v = src2d[j, pl.ds(16 * k, 16)]
        dst1[pl.ds(16 * k, 16)] = jnp.clip(v, 0, _NP - 1)


@functools.partial(
    pl.kernel,
    out_type=jax.ShapeDtypeStruct((2, _NP, 16), jnp.float32),
    mesh=_sc_mesh,
    compiler_params=_sc_params,
    scratch_types=[
        pltpu.VMEM((_DG_SUB, 128), jnp.int32),
        pltpu.VMEM((128,), jnp.int32),
        pltpu.VMEM((128, 16), jnp.float32),
        pltpu.VMEM((128, 16), jnp.float32),
        pltpu.VMEM_SHARED((_NP, 16), jnp.float32),
    ],
)
def _deg_sc(ei_hbm, out_hbm, e_i, ei1, ones_v, zer_v, acc):
    c = lax.axis_index("c")
    s = lax.axis_index("s")

    def _fill(i, _):
        ones_v[i, :] = jnp.ones((16,), jnp.float32)
        zer_v[i, :] = jnp.zeros((16,), jnp.float32)
        return 0

    lax.fori_loop(0, 128, _fill, 0)
    base_row = _RPS * s
    for t in range(_RPS // 128):
        pltpu.sync_copy(zer_v, acc.at[pl.ds(base_row + 128 * t, 128)])
    plsc.subcore_barrier()

    start = _DG_SUB * s
    pltpu.sync_copy(ei_hbm.at[c, pl.ds(start, _DG_SUB)], e_i)

    def _body(j, _):
        _load_idx_chunk(e_i, j, ei1)
        pltpu.sync_copy(ones_v, acc.at[ei1], add=True)
        return 0

    lax.fori_loop(0, _DG_SUB, _body, 0)

    plsc.subcore_barrier()
    pltpu.sync_copy(
        acc.at[pl.ds(base_row, _RPS)],
        out_hbm.at[c, pl.ds(base_row, _RPS)],
    )


@functools.partial(
    pl.kernel,
    out_type=jax.ShapeDtypeStruct((2, _NP, _D), jnp.float32),
    mesh=_sc_mesh,
    compiler_params=_sc_params,
    scratch_types=[
        pltpu.VMEM((_SP_SUB, 128), jnp.int32),
        pltpu.VMEM((_SP_SUB, 128), jnp.int32),
        pltpu.VMEM((128,), jnp.int32),
        pltpu.VMEM((128,), jnp.int32),
        pltpu.VMEM((128, _D), jnp.float32),
        pltpu.VMEM_SHARED((_NP, _D), jnp.float32),
        pltpu.SemaphoreType.DMA,
    ],
)
def _spmm_sc(hs_hbm, src_hbm, dst_hbm, out_hbm, src_i, dst_i, si1, di1,
             rows_v, acc, sem):
    c = lax.axis_index("c")
    s = lax.axis_index("s")

    def _zrow(i, _):
        for j in range(_D // 16):
            rows_v[i, pl.ds(16 * j, 16)] = jnp.zeros((16,), jnp.float32)
        return 0

    lax.fori_loop(0, 128, _zrow, 0)
    base_row = _RPS * s
    for t in range(_RPS // 128):
        pltpu.sync_copy(rows_v, acc.at[pl.ds(base_row + 128 * t, 128)])
    plsc.subcore_barrier()

    start = c * _SP_CORE + _SP_SUB * s
    pltpu.sync_copy(src_hbm.at[pl.ds(start, _SP_SUB)], src_i)
    pltpu.sync_copy(dst_hbm.at[pl.ds(start, _SP_SUB)], dst_i)

    def _body(j, _):
        _load_idx_chunk(src_i, j, si1)
        _load_idx_chunk(dst_i, j, di1)
        pltpu.async_copy(hs_hbm.at[si1], rows_v, sem).wait()
        pltpu.sync_copy(rows_v, acc.at[di1], add=True)
        return 0

    lax.fori_loop(0, _SP_SUB, _body, 0)

    plsc.subcore_barrier()
    pltpu.sync_copy(
        acc.at[pl.ds(base_row, _RPS)],
        out_hbm.at[c, pl.ds(base_row, _RPS)],
    )


_R = 2000


def _stage_a_body(x_ref, tok_ref, dego_ref, m_ref, o_ref):
    ds = lax.rsqrt(jnp.maximum(dego_ref[:, 0:1], 1.0))
    m = m_ref[:]
    o_ref[:] = (x_ref[:] * (1.0 - m) + m * tok_ref[:]) * ds


def _stage_a(x, token, deg_out, m):
    return pl.pallas_call(
        _stage_a_body,
        out_shape=jax.ShapeDtypeStruct((_N, _D), jnp.float32),
        grid=(_N // _R,),
        in_specs=[
            pl.BlockSpec((_R, _D), lambda i: (i, 0)),
            pl.BlockSpec((1, _D), lambda i: (0, 0)),
            pl.BlockSpec((_R, 16), lambda i: (i, 0)),
            pl.BlockSpec((_R, 1), lambda i: (i, 0)),
        ],
        out_specs=pl.BlockSpec((_R, _D), lambda i: (i, 0)),
    )(x, token, deg_out, m)


def _gcn_post(parts, deg_in, deg_out, w, b, g, bet, a):
    agg = parts[0] + parts[1]
    di = lax.rsqrt(jnp.maximum(deg_in[:, 0:1], 1.0))
    ds = lax.rsqrt(jnp.maximum(deg_out[:, 0:1], 1.0))
    y = (jnp.dot(agg, w, preferred_element_type=jnp.float32) + b) * di
    mu = jnp.mean(y, axis=-1, keepdims=True)
    var = jnp.mean((y - mu) ** 2, axis=-1, keepdims=True)
    y = (y - mu) / jnp.sqrt(var + _EPS_LN) * g + bet
    y = jnp.where(y > 0, y, a * y)
    return y, ds


def _stage_b_body(p_ref, degi_ref, dego_ref, w_ref, b_ref, g_ref, bet_ref,
                  a_ref, o_ref):
    y, ds = _gcn_post(p_ref[:], degi_ref[:], dego_ref[:], w_ref[:], b_ref[:],
                      g_ref[:], bet_ref[:], a_ref[0, 0])
    o_ref[:] = y * ds


def _stage_b(parts, deg_in, deg_out, w, b, g, bet, a):
    return pl.pallas_call(
        _stage_b_body,
        out_shape=jax.ShapeDtypeStruct((_N, _D), jnp.float32),
        grid=(_N // _R,),
        in_specs=[
            pl.BlockSpec((2, _R, _D), lambda i: (0, i, 0)),
            pl.BlockSpec((_R, 16), lambda i: (i, 0)),
            pl.BlockSpec((_R, 16), lambda i: (i, 0)),
            pl.BlockSpec((_D, _D), lambda i: (0, 0)),
            pl.BlockSpec((1, _D), lambda i: (0, 0)),
            pl.BlockSpec((1, _D), lambda i: (0, 0)),
            pl.BlockSpec((1, _D), lambda i: (0, 0)),
            pl.BlockSpec((1, 1), lambda i: (0, 0), memory_space=pltpu.SMEM),
        ],
        out_specs=pl.BlockSpec((_R, _D), lambda i: (i, 0)),
    )(parts, deg_in, deg_out, w, b.reshape(1, _D), g.reshape(1, _D),
      bet.reshape(1, _D), a.reshape(1, 1))


def _stage_c_body(p_ref, degi_ref, dego_ref, w_ref, b_ref, g_ref, bet_ref,
                  a_ref, we_ref, m_ref, o_ref):
    y, ds = _gcn_post(p_ref[:], degi_ref[:], dego_ref[:], w_ref[:], b_ref[:],
                      g_ref[:], bet_ref[:], a_ref[0, 0])
    rep = jnp.dot(y, we_ref[:], preferred_element_type=jnp.float32)
    o_ref[:] = rep * (1.0 - m_ref[:]) * ds


def _stage_c(parts, deg_in, deg_out, w, b, g, bet, a, we, m):
    return pl.pallas_call(
        _stage_c_body,
        out_shape=jax.ShapeDtypeStruct((_N, _D), jnp.float32),
        grid=(_N // _R,),
        in_specs=[
            pl.BlockSpec((2, _R, _D), lambda i: (0, i, 0)),
            pl.BlockSpec((_R, 16), lambda i: (i, 0)),
            pl.BlockSpec((_R, 16), lambda i: (i, 0)),
            pl.BlockSpec((_D, _D), lambda i: (0, 0)),
            pl.BlockSpec((1, _D), lambda i: (0, 0)),
            pl.BlockSpec((1, _D), lambda i: (0, 0)),
            pl.BlockSpec((1, _D), lambda i: (0, 0)),
            pl.BlockSpec((1, 1), lambda i: (0, 0), memory_space=pltpu.SMEM),
            pl.BlockSpec((_D, _D), lambda i: (0, 0)),
            pl.BlockSpec((_R, 1), lambda i: (i, 0)),
        ],
        out_specs=pl.BlockSpec((_R, _D), lambda i: (i, 0)),
    )(parts, deg_in, deg_out, w, b.reshape(1, _D), g.reshape(1, _D),
      bet.reshape(1, _D), a.reshape(1, 1), we, m)


def _stage_d_body(p_ref, degi_ref, x_ref, w_ref, b_ref, m_ref, o_ref):
    agg = p_ref[0] + p_ref[1]
    di = lax.rsqrt(jnp.maximum(degi_ref[:, 0:1], 1.0))
    r = (jnp.dot(agg, w_ref[:], preferred_element_type=jnp.float32)
         + b_ref[:]) * di
    xb = x_ref[:]
    nr = jnp.maximum(jnp.sqrt(jnp.sum(r * r, axis=-1, keepdims=True)), 1e-12)
    nx = jnp.maximum(jnp.sqrt(jnp.sum(xb * xb, axis=-1, keepdims=True)), 1e-12)
    cos = jnp.sum(r * xb, axis=-1, keepdims=True) / (nr * nx)
    part = jnp.sum((1.0 - cos) ** 2 * m_ref[:]) * (1.0 / _NUM_MASK)
    prev = jnp.where(pl.program_id(0) == 0, 0.0, o_ref[0, 0])
    o_ref[0, 0] = prev + part


def _stage_d(parts, deg_in, x, w, b, m):
    return pl.pallas_call(
        _stage_d_body,
        out_shape=jax.ShapeDtypeStruct((1, 1), jnp.float32),
        grid=(_N // _R,),
        in_specs=[
            pl.BlockSpec((2, _R, _D), lambda i: (0, i, 0)),
            pl.BlockSpec((_R, 16), lambda i: (i, 0)),
            pl.BlockSpec((_R, _D), lambda i: (i, 0)),
            pl.BlockSpec((_D, _D), lambda i: (0, 0)),
            pl.BlockSpec((1, _D), lambda i: (0, 0)),
            pl.BlockSpec((_R, 1), lambda i: (i, 0)),
        ],
        out_specs=pl.BlockSpec((1, 1), lambda i: (0, 0),
                               memory_space=pltpu.SMEM),
    )(parts, deg_in, x, w, b.reshape(1, _D), m)


def kernel(x, edge_index, W1, b1, g1, beta1, a1, W2, b2, g2, beta2, a2,
           We2d, mask_token, Wd, bd):
    src2d = edge_index[0].reshape(_EC, 128)
    dst2d = edge_index[1].reshape(_EC, 128)

    iota = lax.broadcasted_iota(jnp.int32, (30, 128), 1)
    pad_gather = (iota + 128 * lax.broadcasted_iota(jnp.int32, (30, 128), 0)) % _N
    pad_dummy = _N + (iota % 16)
    half = _EC // 2
    src_sp = jnp.concatenate([src2d[:half], pad_gather, src2d[half:],
                              pad_gather], 0)
    dst_sp = jnp.concatenate([dst2d[:half], pad_dummy, dst2d[half:],
                              pad_dummy], 0)
    pad60 = jnp.concatenate([pad_dummy, pad_dummy], 0)
    ei_dg = jnp.stack([jnp.concatenate([src2d, pad60], 0),
                       jnp.concatenate([dst2d, pad60], 0)])

    m = _mask_vec()
    deg = _deg_sc(ei_dg)
    hs0 = _stage_a(x, mask_token, deg[0], m)
    parts = _spmm_sc(hs0, src_sp, dst_sp)
    hs1 = _stage_b(parts, deg[1], deg[0], W1, b1, g1, beta1, a1)
    parts = _spmm_sc(hs1, src_sp, dst_sp)
    hs2 = _stage_c(parts, deg[1], deg[0], W2, b2, g2, beta2, a2, We2d, m)
    parts = _spmm_sc(hs2, src_sp, dst_sp)
    loss = _stage_d(parts, deg[1], x, Wd, bd, m)
    return loss[0, 0]


# speedup vs baseline: 9.0139x; 1.2614x over previous
import functools

import jax
import jax.numpy as jnp
from jax import lax
from jax.experimental import pallas as pl
from jax.experimental.pallas import tpu as pltpu
import jax.experimental.pallas.tpu_sc as plsc

_N = 10000
_E = 320000
_D = 128
_NUM_MASK = 5000
_EPS_LN = 1e-5

_NP = 10240
_RPS = _NP // 16
_EC = _E // 128
_SP_SUB = 80
_SP_CORE = _SP_SUB * 16
_DG_SUB = 160


def _mask_vec():
    perm = jax.random.permutation(jax.random.key(1), _N)
    return jnp.zeros((_N, 1), jnp.float32).at[perm[:_NUM_MASK], 0].set(1.0)


_sc_mesh = plsc.VectorSubcoreMesh(core_axis_name="c", subcore_axis_name="s")
_sc_params = pltpu.CompilerParams(use_tc_tiling_on_sc=False)


def _load_idx_chunk(src2d, j, dst1):
    for k in range(8):
        v = src2d[j, pl.ds(16 * k, 16)]
        dst1[pl.ds(16 * k, 16)] = jnp.clip(v, 0, _NP - 1)


def _load_idx_half(src2d, row, off, dst1):
    for k in range(4):
        v = src2d[row, pl.ds(off + 16 * k, 16)]
        dst1[pl.ds(16 * k, 16)] = jnp.clip(v, 0, _NP - 1)


@functools.partial(
    pl.kernel,
    out_type=jax.ShapeDtypeStruct((2, _NP, 16), jnp.float32),
    mesh=_sc_mesh,
    compiler_params=_sc_params,
    scratch_types=[
        pltpu.VMEM((_DG_SUB, 128), jnp.int32),
        pltpu.VMEM((128,), jnp.int32),
        pltpu.VMEM((128, 16), jnp.float32),
        pltpu.VMEM((128, 16), jnp.float32),
        pltpu.VMEM_SHARED((_NP, 16), jnp.float32),
    ],
)
def _deg_sc(ei_hbm, out_hbm, e_i, ei1, ones_v, zer_v, acc):
    c = lax.axis_index("c")
    s = lax.axis_index("s")

    def _fill(i, _):
        ones_v[i, :] = jnp.ones((16,), jnp.float32)
        zer_v[i, :] = jnp.zeros((16,), jnp.float32)
        return 0

    lax.fori_loop(0, 128, _fill, 0)
    base_row = _RPS * s
    for t in range(_RPS // 128):
        pltpu.sync_copy(zer_v, acc.at[pl.ds(base_row + 128 * t, 128)])
    plsc.subcore_barrier()

    start = _DG_SUB * s
    pltpu.sync_copy(ei_hbm.at[c, pl.ds(start, _DG_SUB)], e_i)

    def _body(j, _):
        _load_idx_chunk(e_i, j, ei1)
        pltpu.sync_copy(ones_v, acc.at[ei1], add=True)
        return 0

    lax.fori_loop(0, _DG_SUB, _body, 0)

    plsc.subcore_barrier()
    pltpu.sync_copy(
        acc.at[pl.ds(base_row, _RPS)],
        out_hbm.at[c, pl.ds(base_row, _RPS)],
    )


@functools.partial(
    pl.kernel,
    out_type=jax.ShapeDtypeStruct((2, _NP, _D), jnp.float32),
    mesh=_sc_mesh,
    compiler_params=_sc_params,
    scratch_types=[
        pltpu.VMEM((_SP_SUB + 1, 128), jnp.int32),
        pltpu.VMEM((_SP_SUB, 128), jnp.int32),
        pltpu.VMEM((64,), jnp.int32),
        pltpu.VMEM((64,), jnp.int32),
        pltpu.VMEM((64,), jnp.int32),
        pltpu.VMEM((64,), jnp.int32),
        pltpu.VMEM((64, _D), jnp.float32),
        pltpu.VMEM((64, _D), jnp.float32),
        pltpu.VMEM_SHARED((_NP, _D), jnp.float32),
        pltpu.SemaphoreType.DMA,
        pltpu.SemaphoreType.DMA,
    ],
)
def _spmm_sc(hs_hbm, src_hbm, dst_hbm, out_hbm, src_i, dst_i, si_a, si_b,
             di_a, di_b, rows_a, rows_b, acc, sem_a, sem_b):
    c = lax.axis_index("c")
    s = lax.axis_index("s")

    def _zrow(i, _):
        for j in range(_D // 16):
            rows_a[i, pl.ds(16 * j, 16)] = jnp.zeros((16,), jnp.float32)
            rows_b[i, pl.ds(16 * j, 16)] = jnp.zeros((16,), jnp.float32)
        return 0

    lax.fori_loop(0, 64, _zrow, 0)
    base_row = _RPS * s
    for t in range(_RPS // 64):
        buf = rows_a if t % 2 == 0 else rows_b
        pltpu.sync_copy(buf, acc.at[pl.ds(base_row + 64 * t, 64)])
    plsc.subcore_barrier()

    start = c * _SP_CORE + _SP_SUB * s
    pltpu.sync_copy(src_hbm.at[pl.ds(start, _SP_SUB)],
                    src_i.at[pl.ds(0, _SP_SUB)])
    pltpu.sync_copy(dst_hbm.at[pl.ds(start, _SP_SUB)], dst_i)

    _load_idx_half(src_i, 0, 0, si_a)
    pltpu.async_copy(hs_hbm.at[si_a], rows_a, sem_a)

    def _body(t, _):
        _load_idx_half(src_i, t, 64, si_b)
        pltpu.async_copy(hs_hbm.at[si_b], rows_b, sem_b)
        pltpu.make_async_copy(hs_hbm.at[si_a], rows_a, sem_a).wait()
        _load_idx_half(dst_i, t, 0, di_a)
        pltpu.sync_copy(rows_a, acc.at[di_a], add=True)
        _load_idx_half(src_i, t + 1, 0, si_a)
        pltpu.async_copy(hs_hbm.at[si_a], rows_a, sem_a)
        pltpu.make_async_copy(hs_hbm.at[si_b], rows_b, sem_b).wait()
        _load_idx_half(dst_i, t, 64, di_b)
        pltpu.sync_copy(rows_b, acc.at[di_b], add=True)
        return 0

    lax.fori_loop(0, _SP_SUB, _body, 0)
    pltpu.make_async_copy(hs_hbm.at[si_a], rows_a, sem_a).wait()

    plsc.subcore_barrier()
    pltpu.sync_copy(
        acc.at[pl.ds(base_row, _RPS)],
        out_hbm.at[c, pl.ds(base_row, _RPS)],
    )


_R = 2000


def _stage_a_body(x_ref, tok_ref, dego_ref, m_ref, o_ref):
    ds = lax.rsqrt(jnp.maximum(dego_ref[:, 0:1], 1.0))
    m = m_ref[:]
    o_ref[:] = (x_ref[:] * (1.0 - m) + m * tok_ref[:]) * ds


def _stage_a(x, token, deg_out, m):
    return pl.pallas_call(
        _stage_a_body,
        out_shape=jax.ShapeDtypeStruct((_N, _D), jnp.float32),
        grid=(_N // _R,),
        in_specs=[
            pl.BlockSpec((_R, _D), lambda i: (i, 0)),
            pl.BlockSpec((1, _D), lambda i: (0, 0)),
            pl.BlockSpec((_R, 16), lambda i: (i, 0)),
            pl.BlockSpec((_R, 1), lambda i: (i, 0)),
        ],
        out_specs=pl.BlockSpec((_R, _D), lambda i: (i, 0)),
    )(x, token, deg_out, m)


def _gcn_post(parts, deg_in, deg_out, w, b, g, bet, a):
    agg = parts[0] + parts[1]
    di = lax.rsqrt(jnp.maximum(deg_in[:, 0:1], 1.0))
    ds = lax.rsqrt(jnp.maximum(deg_out[:, 0:1], 1.0))
    y = (jnp.dot(agg, w, preferred_element_type=jnp.float32) + b) * di
    mu = jnp.mean(y, axis=-1, keepdims=True)
    var = jnp.mean((y - mu) ** 2, axis=-1, keepdims=True)
    y = (y - mu) / jnp.sqrt(var + _EPS_LN) * g + bet
    y = jnp.where(y > 0, y, a * y)
    return y, ds


def _stage_b_body(p_ref, degi_ref, dego_ref, w_ref, b_ref, g_ref, bet_ref,
                  a_ref, o_ref):
    y, ds = _gcn_post(p_ref[:], degi_ref[:], dego_ref[:], w_ref[:], b_ref[:],
                      g_ref[:], bet_ref[:], a_ref[0, 0])
    o_ref[:] = y * ds


def _stage_b(parts, deg_in, deg_out, w, b, g, bet, a):
    return pl.pallas_call(
        _stage_b_body,
        out_shape=jax.ShapeDtypeStruct((_N, _D), jnp.float32),
        grid=(_N // _R,),
        in_specs=[
            pl.BlockSpec((2, _R, _D), lambda i: (0, i, 0)),
            pl.BlockSpec((_R, 16), lambda i: (i, 0)),
            pl.BlockSpec((_R, 16), lambda i: (i, 0)),
            pl.BlockSpec((_D, _D), lambda i: (0, 0)),
            pl.BlockSpec((1, _D), lambda i: (0, 0)),
            pl.BlockSpec((1, _D), lambda i: (0, 0)),
            pl.BlockSpec((1, _D), lambda i: (0, 0)),
            pl.BlockSpec((1, 1), lambda i: (0, 0), memory_space=pltpu.SMEM),
        ],
        out_specs=pl.BlockSpec((_R, _D), lambda i: (i, 0)),
    )(parts, deg_in, deg_out, w, b.reshape(1, _D), g.reshape(1, _D),
      bet.reshape(1, _D), a.reshape(1, 1))


def _stage_c_body(p_ref, degi_ref, dego_ref, w_ref, b_ref, g_ref, bet_ref,
                  a_ref, we_ref, m_ref, o_ref):
    y, ds = _gcn_post(p_ref[:], degi_ref[:], dego_ref[:], w_ref[:], b_ref[:],
                      g_ref[:], bet_ref[:], a_ref[0, 0])
    rep = jnp.dot(y, we_ref[:], preferred_element_type=jnp.float32)
    o_ref[:] = rep * (1.0 - m_ref[:]) * ds


def _stage_c(parts, deg_in, deg_out, w, b, g, bet, a, we, m):
    return pl.pallas_call(
        _stage_c_body,
        out_shape=jax.ShapeDtypeStruct((_N, _D), jnp.float32),
        grid=(_N // _R,),
        in_specs=[
            pl.BlockSpec((2, _R, _D), lambda i: (0, i, 0)),
            pl.BlockSpec((_R, 16), lambda i: (i, 0)),
            pl.BlockSpec((_R, 16), lambda i: (i, 0)),
            pl.BlockSpec((_D, _D), lambda i: (0, 0)),
            pl.BlockSpec((1, _D), lambda i: (0, 0)),
            pl.BlockSpec((1, _D), lambda i: (0, 0)),
            pl.BlockSpec((1, _D), lambda i: (0, 0)),
            pl.BlockSpec((1, 1), lambda i: (0, 0), memory_space=pltpu.SMEM),
            pl.BlockSpec((_D, _D), lambda i: (0, 0)),
            pl.BlockSpec((_R, 1), lambda i: (i, 0)),
        ],
        out_specs=pl.BlockSpec((_R, _D), lambda i: (i, 0)),
    )(parts, deg_in, deg_out, w, b.reshape(1, _D), g.reshape(1, _D),
      bet.reshape(1, _D), a.reshape(1, 1), we, m)


def _stage_d_body(p_ref, degi_ref, x_ref, w_ref, b_ref, m_ref, o_ref):
    agg = p_ref[0] + p_ref[1]
    di = lax.rsqrt(jnp.maximum(degi_ref[:, 0:1], 1.0))
    r = (jnp.dot(agg, w_ref[:], preferred_element_type=jnp.float32)
         + b_ref[:]) * di
    xb = x_ref[:]
    nr = jnp.maximum(jnp.sqrt(jnp.sum(r * r, axis=-1, keepdims=True)), 1e-12)
    nx = jnp.maximum(jnp.sqrt(jnp.sum(xb * xb, axis=-1, keepdims=True)), 1e-12)
    cos = jnp.sum(r * xb, axis=-1, keepdims=True) / (nr * nx)
    part = jnp.sum((1.0 - cos) ** 2 * m_ref[:]) * (1.0 / _NUM_MASK)
    prev = jnp.where(pl.program_id(0) == 0, 0.0, o_ref[0, 0])
    o_ref[0, 0] = prev + part


def _stage_d(parts, deg_in, x, w, b, m):
    return pl.pallas_call(
        _stage_d_body,
        out_shape=jax.ShapeDtypeStruct((1, 1), jnp.float32),
        grid=(_N // _R,),
        in_specs=[
            pl.BlockSpec((2, _R, _D), lambda i: (0, i, 0)),
            pl.BlockSpec((_R, 16), lambda i: (i, 0)),
            pl.BlockSpec((_R, _D), lambda i: (i, 0)),
            pl.BlockSpec((_D, _D), lambda i: (0, 0)),
            pl.BlockSpec((1, _D), lambda i: (0, 0)),
            pl.BlockSpec((_R, 1), lambda i: (i, 0)),
        ],
        out_specs=pl.BlockSpec((1, 1), lambda i: (0, 0),
                               memory_space=pltpu.SMEM),
    )(parts, deg_in, x, w, b.reshape(1, _D), m)


def kernel(x, edge_index, W1, b1, g1, beta1, a1, W2, b2, g2, beta2, a2,
           We2d, mask_token, Wd, bd):
    src2d = edge_index[0].reshape(_EC, 128)
    dst2d = edge_index[1].reshape(_EC, 128)

    iota = lax.broadcasted_iota(jnp.int32, (30, 128), 1)
    pad_gather = (iota + 128 * lax.broadcasted_iota(jnp.int32, (30, 128), 0)) % _N
    pad_dummy = _N + (iota % 16)
    half = _EC // 2
    src_sp = jnp.concatenate([src2d[:half], pad_gather, src2d[half:],
                              pad_gather], 0)
    dst_sp = jnp.concatenate([dst2d[:half], pad_dummy, dst2d[half:],
                              pad_dummy], 0)
    pad60 = jnp.concatenate([pad_dummy, pad_dummy], 0)
    ei_dg = jnp.stack([jnp.concatenate([src2d, pad60], 0),
                       jnp.concatenate([dst2d, pad60], 0)])

    m = _mask_vec()
    deg = _deg_sc(ei_dg)
    hs0 = _stage_a(x, mask_token, deg[0], m)
    parts = _spmm_sc(hs0, src_sp, dst_sp)
    hs1 = _stage_b(parts, deg[1], deg[0], W1, b1, g1, beta1, a1)
    parts = _spmm_sc(hs1, src_sp, dst_sp)
    hs2 = _stage_c(parts, deg[1], deg[0], W2, b2, g2, beta2, a2, We2d, m)
    parts = _spmm_sc(hs2, src_sp, dst_sp)
    loss = _stage_d(parts, deg[1], x, Wd, bd, m)
    return loss[0, 0]


# speedup vs baseline: 10.1522x; 1.1263x over previous
import functools

import jax
import jax.numpy as jnp
from jax import lax
from jax.experimental import pallas as pl
from jax.experimental.pallas import tpu as pltpu
import jax.experimental.pallas.tpu_sc as plsc

_N = 10000
_E = 320000
_D = 128
_NUM_MASK = 5000
_EPS_LN = 1e-5

_NP = 10240
_RPS = _NP // 16
_EC = _E // 128
_SP_SUB = 80
_SP_CORE = _SP_SUB * 16
_DG_SUB = 160


def _mask_vec():
    perm = jax.random.permutation(jax.random.key(1), _N)
    return jnp.zeros((_N, 1), jnp.float32).at[perm[:_NUM_MASK], 0].set(1.0)


_sc_mesh = plsc.VectorSubcoreMesh(core_axis_name="c", subcore_axis_name="s")
_sc_params = pltpu.CompilerParams(use_tc_tiling_on_sc=False)


def _load_idx_chunk(src2d, j, dst1):
    for k in range(8):
        v = src2d[j, pl.ds(16 * k, 16)]
        dst1[pl.ds(16 * k, 16)] = jnp.clip(v, 0, _NP - 1)


def _load_idx_q(src2d, row, off, dst1):
    for k in range(2):
        v = src2d[row, pl.ds(off + 16 * k, 16)]
        dst1[pl.ds(16 * k, 16)] = jnp.clip(v, 0, _NP - 1)


@functools.partial(
    pl.kernel,
    out_type=jax.ShapeDtypeStruct((2, _NP, 16), jnp.float32),
    mesh=_sc_mesh,
    compiler_params=_sc_params,
    scratch_types=[
        pltpu.VMEM((_DG_SUB, 128), jnp.int32),
        pltpu.VMEM((128,), jnp.int32),
        pltpu.VMEM((128, 16), jnp.float32),
        pltpu.VMEM((128, 16), jnp.float32),
        pltpu.VMEM_SHARED((_NP, 16), jnp.float32),
    ],
)
def _deg_sc(ei_hbm, out_hbm, e_i, ei1, ones_v, zer_v, acc):
    c = lax.axis_index("c")
    s = lax.axis_index("s")

    def _fill(i, _):
        ones_v[i, :] = jnp.ones((16,), jnp.float32)
        zer_v[i, :] = jnp.zeros((16,), jnp.float32)
        return 0

    lax.fori_loop(0, 128, _fill, 0)
    base_row = _RPS * s
    for t in range(_RPS // 128):
        pltpu.sync_copy(zer_v, acc.at[pl.ds(base_row + 128 * t, 128)])
    plsc.subcore_barrier()

    start = _DG_SUB * s
    pltpu.sync_copy(ei_hbm.at[c, pl.ds(start, _DG_SUB)], e_i)

    def _body(j, _):
        _load_idx_chunk(e_i, j, ei1)
        pltpu.sync_copy(ones_v, acc.at[ei1], add=True)
        return 0

    lax.fori_loop(0, _DG_SUB, _body, 0)

    plsc.subcore_barrier()
    pltpu.sync_copy(
        acc.at[pl.ds(base_row, _RPS)],
        out_hbm.at[c, pl.ds(base_row, _RPS)],
    )


@functools.partial(
    pl.kernel,
    out_type=jax.ShapeDtypeStruct((2, _NP, _D), jnp.float32),
    mesh=_sc_mesh,
    compiler_params=_sc_params,
    scratch_types=[
        pltpu.VMEM((_SP_SUB + 1, 128), jnp.int32),
        pltpu.VMEM((_SP_SUB, 128), jnp.int32),
        [pltpu.VMEM((32,), jnp.int32) for _ in range(4)],
        [pltpu.VMEM((32,), jnp.int32) for _ in range(4)],
        [pltpu.VMEM((32, _D), jnp.float32) for _ in range(4)],
        pltpu.VMEM_SHARED((_NP, _D), jnp.float32),
        [pltpu.SemaphoreType.DMA for _ in range(4)],
    ],
)
def _spmm_sc(hs_hbm, src_hbm, dst_hbm, out_hbm, src_i, dst_i, si, di,
             rows, acc, sems):
    c = lax.axis_index("c")
    s = lax.axis_index("s")

    def _zrow(i, _):
        for j in range(_D // 16):
            for q in range(4):
                rows[q][i, pl.ds(16 * j, 16)] = jnp.zeros((16,), jnp.float32)
        return 0

    lax.fori_loop(0, 32, _zrow, 0)
    base_row = _RPS * s
    for t in range(_RPS // 32):
        pltpu.sync_copy(rows[t % 4], acc.at[pl.ds(base_row + 32 * t, 32)])
    plsc.subcore_barrier()

    start = c * _SP_CORE + _SP_SUB * s
    pltpu.sync_copy(src_hbm.at[pl.ds(start, _SP_SUB)],
                    src_i.at[pl.ds(0, _SP_SUB)])
    pltpu.sync_copy(dst_hbm.at[pl.ds(start, _SP_SUB)], dst_i)

    for q in range(4):
        _load_idx_q(src_i, 0, 32 * q, si[q])
        pltpu.async_copy(hs_hbm.at[si[q]], rows[q], sems[q])

    def _body(t, _):
        for q in range(4):
            pltpu.make_async_copy(hs_hbm.at[si[q]], rows[q], sems[q]).wait()
            _load_idx_q(dst_i, t, 32 * q, di[q])
            pltpu.sync_copy(rows[q], acc.at[di[q]], add=True)
            _load_idx_q(src_i, t + 1, 32 * q, si[q])
            pltpu.async_copy(hs_hbm.at[si[q]], rows[q], sems[q])
        return 0

    lax.fori_loop(0, _SP_SUB, _body, 0)
    for q in range(4):
        pltpu.make_async_copy(hs_hbm.at[si[q]], rows[q], sems[q]).wait()

    plsc.subcore_barrier()
    pltpu.sync_copy(
        acc.at[pl.ds(base_row, _RPS)],
        out_hbm.at[c, pl.ds(base_row, _RPS)],
    )


_R = 2000


def _stage_a_body(x_ref, tok_ref, dego_ref, m_ref, o_ref):
    ds = lax.rsqrt(jnp.maximum(dego_ref[:, 0:1], 1.0))
    m = m_ref[:]
    o_ref[:] = (x_ref[:] * (1.0 - m) + m * tok_ref[:]) * ds


def _stage_a(x, token, deg_out, m):
    return pl.pallas_call(
        _stage_a_body,
        out_shape=jax.ShapeDtypeStruct((_N, _D), jnp.float32),
        grid=(_N // _R,),
        in_specs=[
            pl.BlockSpec((_R, _D), lambda i: (i, 0)),
            pl.BlockSpec((1, _D), lambda i: (0, 0)),
            pl.BlockSpec((_R, 16), lambda i: (i, 0)),
            pl.BlockSpec((_R, 1), lambda i: (i, 0)),
        ],
        out_specs=pl.BlockSpec((_R, _D), lambda i: (i, 0)),
    )(x, token, deg_out, m)


def _gcn_post(parts, deg_in, deg_out, w, b, g, bet, a):
    agg = parts[0] + parts[1]
    di = lax.rsqrt(jnp.maximum(deg_in[:, 0:1], 1.0))
    ds = lax.rsqrt(jnp.maximum(deg_out[:, 0:1], 1.0))
    y = (jnp.dot(agg, w, preferred_element_type=jnp.float32) + b) * di
    mu = jnp.mean(y, axis=-1, keepdims=True)
    var = jnp.mean((y - mu) ** 2, axis=-1, keepdims=True)
    y = (y - mu) / jnp.sqrt(var + _EPS_LN) * g + bet
    y = jnp.where(y > 0, y, a * y)
    return y, ds


def _stage_b_body(p_ref, degi_ref, dego_ref, w_ref, b_ref, g_ref, bet_ref,
                  a_ref, o_ref):
    y, ds = _gcn_post(p_ref[:], degi_ref[:], dego_ref[:], w_ref[:], b_ref[:],
                      g_ref[:], bet_ref[:], a_ref[0, 0])
    o_ref[:] = y * ds


def _stage_b(parts, deg_in, deg_out, w, b, g, bet, a):
    return pl.pallas_call(
        _stage_b_body,
        out_shape=jax.ShapeDtypeStruct((_N, _D), jnp.float32),
        grid=(_N // _R,),
        in_specs=[
            pl.BlockSpec((2, _R, _D), lambda i: (0, i, 0)),
            pl.BlockSpec((_R, 16), lambda i: (i, 0)),
            pl.BlockSpec((_R, 16), lambda i: (i, 0)),
            pl.BlockSpec((_D, _D), lambda i: (0, 0)),
            pl.BlockSpec((1, _D), lambda i: (0, 0)),
            pl.BlockSpec((1, _D), lambda i: (0, 0)),
            pl.BlockSpec((1, _D), lambda i: (0, 0)),
            pl.BlockSpec((1, 1), lambda i: (0, 0), memory_space=pltpu.SMEM),
        ],
        out_specs=pl.BlockSpec((_R, _D), lambda i: (i, 0)),
    )(parts, deg_in, deg_out, w, b.reshape(1, _D), g.reshape(1, _D),
      bet.reshape(1, _D), a.reshape(1, 1))


def _stage_c_body(p_ref, degi_ref, dego_ref, w_ref, b_ref, g_ref, bet_ref,
                  a_ref, we_ref, m_ref, o_ref):
    y, ds = _gcn_post(p_ref[:], degi_ref[:], dego_ref[:], w_ref[:], b_ref[:],
                      g_ref[:], bet_ref[:], a_ref[0, 0])
    rep = jnp.dot(y, we_ref[:], preferred_element_type=jnp.float32)
    o_ref[:] = rep * (1.0 - m_ref[:]) * ds


def _stage_c(parts, deg_in, deg_out, w, b, g, bet, a, we, m):
    return pl.pallas_call(
        _stage_c_body,
        out_shape=jax.ShapeDtypeStruct((_N, _D), jnp.float32),
        grid=(_N // _R,),
        in_specs=[
            pl.BlockSpec((2, _R, _D), lambda i: (0, i, 0)),
            pl.BlockSpec((_R, 16), lambda i: (i, 0)),
            pl.BlockSpec((_R, 16), lambda i: (i, 0)),
            pl.BlockSpec((_D, _D), lambda i: (0, 0)),
            pl.BlockSpec((1, _D), lambda i: (0, 0)),
            pl.BlockSpec((1, _D), lambda i: (0, 0)),
            pl.BlockSpec((1, _D), lambda i: (0, 0)),
            pl.BlockSpec((1, 1), lambda i: (0, 0), memory_space=pltpu.SMEM),
            pl.BlockSpec((_D, _D), lambda i: (0, 0)),
            pl.BlockSpec((_R, 1), lambda i: (i, 0)),
        ],
        out_specs=pl.BlockSpec((_R, _D), lambda i: (i, 0)),
    )(parts, deg_in, deg_out, w, b.reshape(1, _D), g.reshape(1, _D),
      bet.reshape(1, _D), a.reshape(1, 1), we, m)


def _stage_d_body(p_ref, degi_ref, x_ref, w_ref, b_ref, m_ref, o_ref):
    agg = p_ref[0] + p_ref[1]
    di = lax.rsqrt(jnp.maximum(degi_ref[:, 0:1], 1.0))
    r = (jnp.dot(agg, w_ref[:], preferred_element_type=jnp.float32)
         + b_ref[:]) * di
    xb = x_ref[:]
    nr = jnp.maximum(jnp.sqrt(jnp.sum(r * r, axis=-1, keepdims=True)), 1e-12)
    nx = jnp.maximum(jnp.sqrt(jnp.sum(xb * xb, axis=-1, keepdims=True)), 1e-12)
    cos = jnp.sum(r * xb, axis=-1, keepdims=True) / (nr * nx)
    part = jnp.sum((1.0 - cos) ** 2 * m_ref[:]) * (1.0 / _NUM_MASK)
    prev = jnp.where(pl.program_id(0) == 0, 0.0, o_ref[0, 0])
    o_ref[0, 0] = prev + part


def _stage_d(parts, deg_in, x, w, b, m):
    return pl.pallas_call(
        _stage_d_body,
        out_shape=jax.ShapeDtypeStruct((1, 1), jnp.float32),
        grid=(_N // _R,),
        in_specs=[
            pl.BlockSpec((2, _R, _D), lambda i: (0, i, 0)),
            pl.BlockSpec((_R, 16), lambda i: (i, 0)),
            pl.BlockSpec((_R, _D), lambda i: (i, 0)),
            pl.BlockSpec((_D, _D), lambda i: (0, 0)),
            pl.BlockSpec((1, _D), lambda i: (0, 0)),
            pl.BlockSpec((_R, 1), lambda i: (i, 0)),
        ],
        out_specs=pl.BlockSpec((1, 1), lambda i: (0, 0),
                               memory_space=pltpu.SMEM),
    )(parts, deg_in, x, w, b.reshape(1, _D), m)


def kernel(x, edge_index, W1, b1, g1, beta1, a1, W2, b2, g2, beta2, a2,
           We2d, mask_token, Wd, bd):
    src2d = edge_index[0].reshape(_EC, 128)
    dst2d = edge_index[1].reshape(_EC, 128)

    iota = lax.broadcasted_iota(jnp.int32, (30, 128), 1)
    pad_gather = (iota + 128 * lax.broadcasted_iota(jnp.int32, (30, 128), 0)) % _N
    pad_dummy = _N + (iota % 16)
    half = _EC // 2
    src_sp = jnp.concatenate([src2d[:half], pad_gather, src2d[half:],
                              pad_gather], 0)
    dst_sp = jnp.concatenate([dst2d[:half], pad_dummy, dst2d[half:],
                              pad_dummy], 0)
    pad60 = jnp.concatenate([pad_dummy, pad_dummy], 0)
    ei_dg = jnp.stack([jnp.concatenate([src2d, pad60], 0),
                       jnp.concatenate([dst2d, pad60], 0)])

    m = _mask_vec()
    deg = _deg_sc(ei_dg)
    hs0 = _stage_a(x, mask_token, deg[0], m)
    parts = _spmm_sc(hs0, src_sp, dst_sp)
    hs1 = _stage_b(parts, deg[1], deg[0], W1, b1, g1, beta1, a1)
    parts = _spmm_sc(hs1, src_sp, dst_sp)
    hs2 = _stage_c(parts, deg[1], deg[0], W2, b2, g2, beta2, a2, We2d, m)
    parts = _spmm_sc(hs2, src_sp, dst_sp)
    loss = _stage_d(parts, deg[1], x, Wd, bd, m)
    return loss[0, 0]
